# baseline (speedup 1.0000x reference)
import jax
import jax.numpy as jnp
from jax import lax
from jax.experimental import pallas as pl
from jax.experimental.pallas import tpu as pltpu
from jax.experimental.pallas import tpu_sc as plsc

BATCH = 16384
HIST = 200
D = 64
LANES = 16
ROWS = 1000000
DW = D // 2

RBLK = 128
NBLK = ROWS // RBLK
TAILR = NBLK * RBLK
TAILN = ROWS - TAILR
NW = 32
BLK_LO = NBLK // NW
BLK_EXTRA = NBLK % NW

EPW = BATCH // NW
G = 4
NG = EPW // G
NGP = NG // 2
IDX_PER_G = G * HIST
CHUNK = 80
NCHUNK = IDX_PER_G // CHUNK
OUT_BUF = 64
GPF = OUT_BUF // G


def _encode_body(tt_ref, tail_ref, out_ref, in_a, in_b, enc_a, enc_b,
                 sem_ia, sem_ib, sem_oa, sem_ob):
    nc = 2
    wid = lax.axis_index("s") * nc + lax.axis_index("c")
    base = wid * BLK_LO + jnp.minimum(wid, BLK_EXTRA)
    cnt = BLK_LO + jnp.where(wid < BLK_EXTRA, 1, 0)
    rows16 = [lax.iota(jnp.int32, LANES) + 16 * c for c in range(4)]

    def fire_in(brel, in_v, sem):
        pltpu.async_copy(
            tt_ref.at[:, pl.ds((base + brel) * RBLK, RBLK)], in_v, sem)

    def wait_in(brel, in_v, sem):
        pltpu.make_async_copy(
            tt_ref.at[:, pl.ds((base + brel) * RBLK, RBLK)], in_v, sem).wait()

    def fire_out(brel, enc_v, sem):
        pltpu.async_copy(
            enc_v, out_ref.at[pl.ds((base + brel) * RBLK * DW, RBLK * DW)],
            sem)

    def wait_out(brel, enc_v, sem):
        pltpu.make_async_copy(
            enc_v, out_ref.at[pl.ds((base + brel) * RBLK * DW, RBLK * DW)],
            sem).wait()

    def encode_block(in_v, enc_v):
        for r in range(RBLK):
            col = jnp.full((LANES,), r, dtype=jnp.int32)
            v = [plsc.load_gather(in_v, [rows16[c], col]) for c in range(4)]
            pk0 = plsc.pack(v[0], v[1], format=plsc.PackFormat.INTERLEAVED)
            pk1 = plsc.pack(v[2], v[3], format=plsc.PackFormat.INTERLEAVED)
            enc_v[pl.ds(r * DW, LANES)] = plsc.bitcast(pk0, jnp.int32)
            enc_v[pl.ds(r * DW + LANES, LANES)] = plsc.bitcast(pk1, jnp.int32)

    @pl.when(cnt > 0)
    def _prime():
        fire_in(0, in_a, sem_ia)

    def pair_body(i, carry):
        b0 = 2 * i
        b1 = 2 * i + 1

        @pl.when(b1 < cnt)
        def _fire_b():
            fire_in(b1, in_b, sem_ib)

        @pl.when(b0 < cnt)
        def _do_a():
            wait_in(b0, in_a, sem_ia)

            @pl.when(i > 0)
            def _():
                wait_out(b0 - 2, enc_a, sem_oa)

            encode_block(in_a, enc_a)
            fire_out(b0, enc_a, sem_oa)

        @pl.when(b0 + 2 < cnt)
        def _refire_a():
            fire_in(b0 + 2, in_a, sem_ia)

        @pl.when(b1 < cnt)
        def _do_b():
            wait_in(b1, in_b, sem_ib)

            @pl.when(i > 0)
            def _():
                wait_out(b1 - 2, enc_b, sem_ob)

            encode_block(in_b, enc_b)
            fire_out(b1, enc_b, sem_ob)

        return carry

    lax.fori_loop(0, (BLK_LO + 2) // 2, pair_body, 0)

    last_a = ((cnt - 1) // 2) * 2
    last_b = ((cnt - 2) // 2) * 2 + 1

    @pl.when(cnt > 0)
    def _drain_a():
        wait_out(last_a, enc_a, sem_oa)

    @pl.when(cnt > 1)
    def _drain_b():
        wait_out(last_b, enc_b, sem_ob)

    @pl.when(wid == 0)
    def _tail():
        pltpu.sync_copy(tail_ref, out_ref.at[pl.ds(TAILR * DW, TAILN * DW)])


def _gather_body(x_ref, table_ref, out_ref, idx_a, idx_b, rows_a, rows_b,
                 out_v, sem_a, sem_b):
    nc = 2
    wid = lax.axis_index("s") * nc + lax.axis_index("c")
    base_elem = wid * EPW
    scale = jnp.full((LANES,), 1.0 / HIST, dtype=jnp.float32)

    def stage_idx(g, idx_v):
        pltpu.sync_copy(
            x_ref.at[pl.ds((base_elem + g * G) * HIST, IDX_PER_G)], idx_v)

    def fire(idx_v, rows_v, sem):
        for k in range(NCHUNK):
            pltpu.async_copy(
                table_ref.at[idx_v.at[pl.ds(k * CHUNK, CHUNK)]],
                rows_v.at[pl.ds(k * CHUNK, CHUNK), :],
                sem)

    def drain(idx_v, rows_v, sem):
        for k in range(NCHUNK):
            pltpu.make_async_copy(
                table_ref.at[idx_v.at[pl.ds(k * CHUNK, CHUNK)]],
                rows_v.at[pl.ds(k * CHUNK, CHUNK), :],
                sem).wait()

    def reduce(g, rows_v):
        orow0 = (g % GPF) * G
        for e in range(G):
            rb = e * HIST

            def red_body(j, accs):
                r0 = rb + j * 8
                a0, a1, a2, a3 = accs
                for u in range(8):
                    ab0 = plsc.bitcast(rows_v[r0 + u, pl.ds(0, LANES)],
                                       jnp.bfloat16)
                    ab1 = plsc.bitcast(rows_v[r0 + u, pl.ds(LANES, LANES)],
                                       jnp.bfloat16)
                    x0, y0 = plsc.unpack(ab0, format=plsc.PackFormat.INTERLEAVED)
                    x1, y1 = plsc.unpack(ab1, format=plsc.PackFormat.INTERLEAVED)
                    a0 = a0 + x0
                    a1 = a1 + y0
                    a2 = a2 + x1
                    a3 = a3 + y1
                return (a0, a1, a2, a3)

            z = jnp.zeros((LANES,), jnp.float32)
            accs = lax.fori_loop(0, HIST // 8, red_body, (z,) * 4)
            for c in range(4):
                out_v[orow0 + e, pl.ds(c * LANES, LANES)] = accs[c] * scale

    stage_idx(0, idx_a)
    fire(idx_a, rows_a, sem_a)

    def pair_body(i, carry):
        g0 = 2 * i
        g1 = 2 * i + 1

        stage_idx(g1, idx_b)
        fire(idx_b, rows_b, sem_b)

        drain(idx_a, rows_a, sem_a)
        reduce(g0, rows_a)

        @pl.when(i < NGP - 1)
        def _refire_a():
            stage_idx(g0 + 2, idx_a)
            fire(idx_a, rows_a, sem_a)

        drain(idx_b, rows_b, sem_b)
        reduce(g1, rows_b)

        @pl.when(i % (GPF // 2) == GPF // 2 - 1)
        def _flush():
            ob = base_elem + (g1 // GPF) * OUT_BUF
            pltpu.sync_copy(out_v, out_ref.at[pl.ds(ob, OUT_BUF), :])

        return carry

    lax.fori_loop(0, NGP, pair_body, 0)


def kernel(x, table):
    xf = x.reshape(-1).astype(jnp.int32)
    tt = table.T
    th = table[TAILR:].astype(jnp.bfloat16)
    tail = th.reshape(TAILN, 2, 2, LANES).transpose(0, 1, 3, 2).reshape(-1)
    tail = lax.bitcast_convert_type(tail.reshape(-1, 2), jnp.int32)

    mesh = plsc.VectorSubcoreMesh(core_axis_name="c", subcore_axis_name="s")
    encode = pl.kernel(
        _encode_body,
        out_type=jax.ShapeDtypeStruct((ROWS * DW,), jnp.int32),
        mesh=mesh,
        scratch_types=[
            pltpu.VMEM((D, RBLK), jnp.float32),
            pltpu.VMEM((D, RBLK), jnp.float32),
            pltpu.VMEM((RBLK * DW,), jnp.int32),
            pltpu.VMEM((RBLK * DW,), jnp.int32),
            pltpu.SemaphoreType.DMA,
            pltpu.SemaphoreType.DMA,
            pltpu.SemaphoreType.DMA,
            pltpu.SemaphoreType.DMA,
        ],
        compiler_params=pltpu.CompilerParams(
            use_tc_tiling_on_sc=True, needs_layout_passes=False),
    )
    enc = encode(tt, tail)

    gather = pl.kernel(
        _gather_body,
        out_type=jax.ShapeDtypeStruct((BATCH, D), jnp.float32),
        mesh=mesh,
        scratch_types=[
            pltpu.VMEM((IDX_PER_G,), jnp.int32),
            pltpu.VMEM((IDX_PER_G,), jnp.int32),
            pltpu.VMEM((IDX_PER_G, DW), jnp.int32),
            pltpu.VMEM((IDX_PER_G, DW), jnp.int32),
            pltpu.VMEM((OUT_BUF, D), jnp.float32),
            pltpu.SemaphoreType.DMA,
            pltpu.SemaphoreType.DMA,
        ],
        compiler_params=pltpu.CompilerParams(
            use_tc_tiling_on_sc=False, needs_layout_passes=False),
    )
    return gather(xf, enc.reshape(ROWS, DW))


# speedup vs baseline: 1.0018x; 1.0018x over previous
import jax
import jax.numpy as jnp
from jax import lax
from jax.experimental import pallas as pl
from jax.experimental.pallas import tpu as pltpu
from jax.experimental.pallas import tpu_sc as plsc

BATCH = 16384
HIST = 200
D = 64
LANES = 16
ROWS = 1000000
DW = D // 2

RBLK = 128
INSTR = 129
NBLK = ROWS // RBLK
TAILR = NBLK * RBLK
TAILN = ROWS - TAILR
NW = 32
BLK_LO = NBLK // NW
BLK_EXTRA = NBLK % NW

EPW = BATCH // NW
G = 4
NG = EPW // G
NGP = NG // 2
IDX_PER_G = G * HIST
CHUNK = 80
NCHUNK = IDX_PER_G // CHUNK
OUT_BUF = 64
GPF = OUT_BUF // G


def _encode_body(tt_ref, tail_ref, out_ref, in_a, in_b, enc_a, enc_b,
                 sem_ia, sem_ib, sem_oa, sem_ob):
    nc = 2
    wid = lax.axis_index("s") * nc + lax.axis_index("c")
    base = wid * BLK_LO + jnp.minimum(wid, BLK_EXTRA)
    cnt = BLK_LO + jnp.where(wid < BLK_EXTRA, 1, 0)
    rows16 = [lax.iota(jnp.int32, LANES) + 16 * c for c in range(4)]

    def fire_in(brel, in_v, sem):
        pltpu.async_copy(
            tt_ref.at[:, pl.ds((base + brel) * RBLK, RBLK)],
            in_v.at[:, pl.ds(0, RBLK)], sem)

    def wait_in(brel, in_v, sem):
        pltpu.make_async_copy(
            tt_ref.at[:, pl.ds((base + brel) * RBLK, RBLK)],
            in_v.at[:, pl.ds(0, RBLK)], sem).wait()

    def fire_out(brel, enc_v, sem):
        pltpu.async_copy(
            enc_v, out_ref.at[pl.ds((base + brel) * RBLK * DW, RBLK * DW)],
            sem)

    def wait_out(brel, enc_v, sem):
        pltpu.make_async_copy(
            enc_v, out_ref.at[pl.ds((base + brel) * RBLK * DW, RBLK * DW)],
            sem).wait()

    def encode_block(in_v, enc_v):
        def row8_body(jj, carry2):
            for u in range(8):
                r = jj * 8 + u
                col = jnp.zeros((LANES,), jnp.int32) + r
                v = [plsc.load_gather(in_v, [rows16[c], col]) for c in range(4)]
                pk0 = plsc.pack(v[0], v[1], format=plsc.PackFormat.INTERLEAVED)
                pk1 = plsc.pack(v[2], v[3], format=plsc.PackFormat.INTERLEAVED)
                enc_v[pl.ds(r * DW, LANES)] = plsc.bitcast(pk0, jnp.int32)
                enc_v[pl.ds(r * DW + LANES, LANES)] = plsc.bitcast(pk1, jnp.int32)
            return carry2

        lax.fori_loop(0, RBLK // 8, row8_body, 0)

    @pl.when(cnt > 0)
    def _prime():
        fire_in(0, in_a, sem_ia)

    def pair_body(i, carry):
        b0 = 2 * i
        b1 = 2 * i + 1

        @pl.when(b1 < cnt)
        def _fire_b():
            fire_in(b1, in_b, sem_ib)

        @pl.when(b0 < cnt)
        def _do_a():
            wait_in(b0, in_a, sem_ia)

            @pl.when(i > 0)
            def _():
                wait_out(b0 - 2, enc_a, sem_oa)

            encode_block(in_a, enc_a)
            fire_out(b0, enc_a, sem_oa)

        @pl.when(b0 + 2 < cnt)
        def _refire_a():
            fire_in(b0 + 2, in_a, sem_ia)

        @pl.when(b1 < cnt)
        def _do_b():
            wait_in(b1, in_b, sem_ib)

            @pl.when(i > 0)
            def _():
                wait_out(b1 - 2, enc_b, sem_ob)

            encode_block(in_b, enc_b)
            fire_out(b1, enc_b, sem_ob)

        return carry

    lax.fori_loop(0, (BLK_LO + 2) // 2, pair_body, 0)

    last_a = ((cnt - 1) // 2) * 2
    last_b = ((cnt - 2) // 2) * 2 + 1

    @pl.when(cnt > 0)
    def _drain_a():
        wait_out(last_a, enc_a, sem_oa)

    @pl.when(cnt > 1)
    def _drain_b():
        wait_out(last_b, enc_b, sem_ob)

    @pl.when(wid == 0)
    def _tail():
        pltpu.sync_copy(tail_ref, out_ref.at[pl.ds(TAILR * DW, TAILN * DW)])


def _gather_body(x_ref, table_ref, out_ref, idx_a, idx_b, rows_a, rows_b,
                 out_v, sem_a, sem_b):
    nc = 2
    wid = lax.axis_index("s") * nc + lax.axis_index("c")
    base_elem = wid * EPW
    scale = jnp.full((LANES,), 1.0 / HIST, dtype=jnp.float32)

    def stage_idx(g, idx_v):
        pltpu.sync_copy(
            x_ref.at[pl.ds((base_elem + g * G) * HIST, IDX_PER_G)], idx_v)

    def fire(idx_v, rows_v, sem):
        for k in range(NCHUNK):
            pltpu.async_copy(
                table_ref.at[idx_v.at[pl.ds(k * CHUNK, CHUNK)]],
                rows_v.at[pl.ds(k * CHUNK, CHUNK), :],
                sem)

    def drain(idx_v, rows_v, sem):
        for k in range(NCHUNK):
            pltpu.make_async_copy(
                table_ref.at[idx_v.at[pl.ds(k * CHUNK, CHUNK)]],
                rows_v.at[pl.ds(k * CHUNK, CHUNK), :],
                sem).wait()

    def reduce(g, rows_v):
        orow0 = (g % GPF) * G
        for e in range(G):
            rb = e * HIST

            def red_body(j, accs):
                r0 = rb + j * 8
                a0, a1, a2, a3 = accs
                for u in range(8):
                    ab0 = plsc.bitcast(rows_v[r0 + u, pl.ds(0, LANES)],
                                       jnp.bfloat16)
                    ab1 = plsc.bitcast(rows_v[r0 + u, pl.ds(LANES, LANES)],
                                       jnp.bfloat16)
                    x0, y0 = plsc.unpack(ab0, format=plsc.PackFormat.INTERLEAVED)
                    x1, y1 = plsc.unpack(ab1, format=plsc.PackFormat.INTERLEAVED)
                    a0 = a0 + x0
                    a1 = a1 + y0
                    a2 = a2 + x1
                    a3 = a3 + y1
                return (a0, a1, a2, a3)

            z = jnp.zeros((LANES,), jnp.float32)
            accs = lax.fori_loop(0, HIST // 8, red_body, (z,) * 4)
            for c in range(4):
                out_v[orow0 + e, pl.ds(c * LANES, LANES)] = accs[c] * scale

    stage_idx(0, idx_a)
    fire(idx_a, rows_a, sem_a)

    def pair_body(i, carry):
        g0 = 2 * i
        g1 = 2 * i + 1

        stage_idx(g1, idx_b)
        fire(idx_b, rows_b, sem_b)

        drain(idx_a, rows_a, sem_a)
        reduce(g0, rows_a)

        @pl.when(i < NGP - 1)
        def _refire_a():
            stage_idx(g0 + 2, idx_a)
            fire(idx_a, rows_a, sem_a)

        drain(idx_b, rows_b, sem_b)
        reduce(g1, rows_b)

        @pl.when(i % (GPF // 2) == GPF // 2 - 1)
        def _flush():
            ob = base_elem + (g1 // GPF) * OUT_BUF
            pltpu.sync_copy(out_v, out_ref.at[pl.ds(ob, OUT_BUF), :])

        return carry

    lax.fori_loop(0, NGP, pair_body, 0)


def kernel(x, table):
    xf = x.reshape(-1).astype(jnp.int32)
    tt = table.T
    th = table[TAILR:].astype(jnp.bfloat16)
    tail = th.reshape(TAILN, 2, 2, LANES).transpose(0, 1, 3, 2).reshape(-1)
    tail = lax.bitcast_convert_type(tail.reshape(-1, 2), jnp.int32)

    mesh = plsc.VectorSubcoreMesh(core_axis_name="c", subcore_axis_name="s")
    encode = pl.kernel(
        _encode_body,
        out_type=jax.ShapeDtypeStruct((ROWS * DW,), jnp.int32),
        mesh=mesh,
        scratch_types=[
            pltpu.VMEM((D, INSTR), jnp.float32),
            pltpu.VMEM((D, INSTR), jnp.float32),
            pltpu.VMEM((RBLK * DW,), jnp.int32),
            pltpu.VMEM((RBLK * DW,), jnp.int32),
            pltpu.SemaphoreType.DMA,
            pltpu.SemaphoreType.DMA,
            pltpu.SemaphoreType.DMA,
            pltpu.SemaphoreType.DMA,
        ],
        compiler_params=pltpu.CompilerParams(
            use_tc_tiling_on_sc=True, needs_layout_passes=False),
    )
    enc = encode(tt, tail)

    gather = pl.kernel(
        _gather_body,
        out_type=jax.ShapeDtypeStruct((BATCH, D), jnp.float32),
        mesh=mesh,
        scratch_types=[
            pltpu.VMEM((IDX_PER_G,), jnp.int32),
            pltpu.VMEM((IDX_PER_G,), jnp.int32),
            pltpu.VMEM((IDX_PER_G, DW), jnp.int32),
            pltpu.VMEM((IDX_PER_G, DW), jnp.int32),
            pltpu.VMEM((OUT_BUF, D), jnp.float32),
            pltpu.SemaphoreType.DMA,
            pltpu.SemaphoreType.DMA,
        ],
        compiler_params=pltpu.CompilerParams(
            use_tc_tiling_on_sc=False, needs_layout_passes=False),
    )
    return gather(xf, enc.reshape(ROWS, DW))


# speedup vs baseline: 1.5751x; 1.5722x over previous
import jax
import jax.numpy as jnp
from jax import lax
from jax.experimental import pallas as pl
from jax.experimental.pallas import tpu as pltpu
from jax.experimental.pallas import tpu_sc as plsc

BATCH = 16384
HIST = 200
D = 64
LANES = 16
NCOL = D // LANES

NW = 32
EPW = BATCH // NW
G = 4
NG = EPW // G
NGP = NG // 2
IDX_PER_G = G * HIST
CHUNK = 80
NCHUNK = IDX_PER_G // CHUNK
OUT_BUF = 64
GPF = OUT_BUF // G


def _body(x_ref, table_ref, out_ref, idx_a, idx_b, rows_a, rows_b, out_v,
          sem_a, sem_b):
    nc = 2
    wid = lax.axis_index("s") * nc + lax.axis_index("c")
    base_elem = wid * EPW
    scale = jnp.full((LANES,), 1.0 / HIST, dtype=jnp.float32)

    def stage_idx(g, idx_v):
        pltpu.sync_copy(
            x_ref.at[pl.ds((base_elem + g * G) * HIST, IDX_PER_G)], idx_v)

    def fire(idx_v, rows_v, sem):
        for k in range(NCHUNK):
            pltpu.async_copy(
                table_ref.at[idx_v.at[pl.ds(k * CHUNK, CHUNK)]],
                rows_v.at[pl.ds(k * CHUNK, CHUNK), :],
                sem)

    def drain(idx_v, rows_v, sem):
        for k in range(NCHUNK):
            pltpu.make_async_copy(
                table_ref.at[idx_v.at[pl.ds(k * CHUNK, CHUNK)]],
                rows_v.at[pl.ds(k * CHUNK, CHUNK), :],
                sem).wait()

    def reduce(g, rows_v):
        orow0 = (g % GPF) * G
        for e in range(G):
            rb = e * HIST

            def red_body(j, accs):
                r0 = rb + j * 8
                new = list(accs)
                for u in range(8):
                    for c in range(NCOL):
                        new[c] = new[c] + rows_v[r0 + u, pl.ds(c * LANES, LANES)]
                return tuple(new)

            z = jnp.zeros((LANES,), jnp.float32)
            accs = lax.fori_loop(0, HIST // 8, red_body, (z,) * NCOL)
            for c in range(NCOL):
                out_v[orow0 + e, pl.ds(c * LANES, LANES)] = accs[c] * scale

    stage_idx(0, idx_a)
    fire(idx_a, rows_a, sem_a)

    def pair_body(i, carry):
        g0 = 2 * i
        g1 = 2 * i + 1

        stage_idx(g1, idx_b)
        fire(idx_b, rows_b, sem_b)

        drain(idx_a, rows_a, sem_a)
        reduce(g0, rows_a)

        @pl.when(i < NGP - 1)
        def _refire_a():
            stage_idx(g0 + 2, idx_a)
            fire(idx_a, rows_a, sem_a)

        drain(idx_b, rows_b, sem_b)
        reduce(g1, rows_b)

        @pl.when(i % (GPF // 2) == GPF // 2 - 1)
        def _flush():
            ob = base_elem + (g1 // GPF) * OUT_BUF
            pltpu.sync_copy(out_v, out_ref.at[pl.ds(ob, OUT_BUF), :])

        return carry

    lax.fori_loop(0, NGP, pair_body, 0)


def kernel(x, table):
    xf = x.reshape(-1).astype(jnp.int32)
    mesh = plsc.VectorSubcoreMesh(core_axis_name="c", subcore_axis_name="s")
    f = pl.kernel(
        _body,
        out_type=jax.ShapeDtypeStruct((BATCH, D), jnp.float32),
        mesh=mesh,
        scratch_types=[
            pltpu.VMEM((IDX_PER_G,), jnp.int32),
            pltpu.VMEM((IDX_PER_G,), jnp.int32),
            pltpu.VMEM((IDX_PER_G, D), jnp.float32),
            pltpu.VMEM((IDX_PER_G, D), jnp.float32),
            pltpu.VMEM((OUT_BUF, D), jnp.float32),
            pltpu.SemaphoreType.DMA,
            pltpu.SemaphoreType.DMA,
        ],
        compiler_params=pltpu.CompilerParams(use_tc_tiling_on_sc=False),
    )
    return f(xf, table)
